# baseline (speedup 1.0000x reference)
import functools

import jax
import jax.numpy as jnp
from jax import lax
from jax.experimental import pallas as pl
from jax.experimental.pallas import tpu as pltpu
from jax.experimental.pallas import tpu_sc as plsc

_B, _T, _LO, _LA = 256, 50, 16, 4
_LT = _LO + 1 + _LA
_D = 64
_NW = 32
_NPOS = _T * _LT
_BLK = _B * _D
_N_OBS = _T * _LO
_N_ACT = _T * _LA
_OBS_PER_W = _N_OBS // _NW

_mesh = plsc.VectorSubcoreMesh(core_axis_name="c", subcore_axis_name="s")


@functools.partial(
    pl.kernel,
    mesh=_mesh,
    compiler_params=pltpu.CompilerParams(use_tc_tiling_on_sc=False,
                                         needs_layout_passes=False),
    out_type=jax.ShapeDtypeStruct((_NPOS, _BLK), jnp.float32),
    scratch_types=[
        pltpu.VMEM((2, _B), jnp.int32),
        pltpu.VMEM((2, _B, _D), jnp.float32),
        pltpu.VMEM((2, _BLK), jnp.float32),
        pltpu.VMEM((_D,), jnp.float32),
        pltpu.VMEM((_LO, _D), jnp.float32),
        pltpu.VMEM((_LA, _D), jnp.float32),
        pltpu.VMEM((_T, _D), jnp.float32),
        pltpu.VMEM((1, _D), jnp.float32),
        pltpu.VMEM((1, _D), jnp.float32),
        pltpu.SemaphoreType.DMA,
        pltpu.SemaphoreType.DMA,
        pltpu.SemaphoreType.DMA,
        pltpu.SemaphoreType.DMA,
    ],
)
def _episode_sc(obs_idx_hbm, act_idx_hbm, w_obs_hbm, w_act_hbm, w_sp_hbm,
                pe_obs_hbm, pe_act_hbm, pe_sp_hbm, pe_time_hbm, out_hbm,
                idx_v, rows_v, out_v, bias_v,
                pe_obs_v, pe_act_v, pe_time_v, sp_v, pesp_v,
                sem_g0, sem_g1, sem_out0, sem_out1):
    wid = lax.axis_index("s") * 2 + lax.axis_index("c")
    iota16 = lax.iota(jnp.int32, 16)
    sem_g = [sem_g0, sem_g1]
    sem_out = [sem_out0, sem_out1]

    pltpu.sync_copy(pe_obs_hbm.at[pl.ds(0, _LO)], pe_obs_v)
    pltpu.sync_copy(pe_act_hbm, pe_act_v)
    pltpu.sync_copy(pe_time_hbm, pe_time_v)
    pltpu.sync_copy(w_sp_hbm.at[pl.ds(0, 1)], sp_v)
    pltpu.sync_copy(pe_sp_hbm, pesp_v)
    for c in range(_D // 16):
        sp_v[0, pl.ds(c * 16, 16)] = (
            sp_v[0, pl.ds(c * 16, 16)] + pesp_v[0, pl.ds(c * 16, 16)]
        )

    def gather_descs(table_hbm, slot):
        d0 = pltpu.make_async_copy(
            table_hbm.at[idx_v.at[slot, pl.ds(0, 128)]],
            rows_v.at[slot, pl.ds(0, 128)], sem_g[slot])
        d1 = pltpu.make_async_copy(
            table_hbm.at[idx_v.at[slot, pl.ds(128, 128)]],
            rows_v.at[slot, pl.ds(128, 128)], sem_g[slot])
        return d0, d1

    def fire(tok_off, half_stride, tok_hbm, table_hbm, slot):
        pltpu.sync_copy(tok_hbm.at[pl.ds(tok_off, 128)],
                        idx_v.at[slot, pl.ds(0, 128)])
        pltpu.sync_copy(tok_hbm.at[pl.ds(tok_off + half_stride, 128)],
                        idx_v.at[slot, pl.ds(128, 128)])
        d0, d1 = gather_descs(table_hbm, slot)
        d0.start()
        d1.start()

    def wait_gathers(table_hbm, slot):
        d0, d1 = gather_descs(table_hbm, slot)
        d0.wait()
        d1.wait()

    def out_desc(pos, slot):
        return pltpu.make_async_copy(out_v.at[slot], out_hbm.at[pos],
                                     sem_out[slot])

    def transpose_bias_block(slot):
        rows = rows_v.at[slot]

        def d_body(d, carry):
            d_vec = jnp.broadcast_to(d, (16,)).astype(jnp.int32)
            bias_d = plsc.load_gather(bias_v, [d_vec])
            obase = lax.div(d, 8) * 2048 + lax.rem(d, 8) * 128
            for bb in range(16):
                row_idx = iota16 + (bb * 16)
                v = plsc.load_gather(rows, [row_idx, d_vec])
                off = obase + (bb // 8) * 1024 + (bb % 8) * 16
                out_v[slot, pl.ds(off, 16)] = v + bias_d
            return carry

        lax.fori_loop(0, _D, d_body, 0)

    def splat_block(slot):
        def d_body(d, carry):
            d_vec = jnp.broadcast_to(d, (16,)).astype(jnp.int32)
            bias_d = plsc.load_gather(bias_v, [d_vec])
            obase = lax.div(d, 8) * 2048 + lax.rem(d, 8) * 128
            for bb in range(16):
                off = obase + (bb // 8) * 1024 + (bb % 8) * 16
                out_v[slot, pl.ds(off, 16)] = bias_d
            return carry

        lax.fori_loop(0, _D, d_body, 0)

    def obs_unit(k):
        i = wid + _NW * k
        t = lax.div(i, _LO)
        s = lax.rem(i, _LO)
        tok_off = t * 4096 + lax.div(s, 8) * 2048 + lax.rem(s, 8) * 128
        pos = t * _LT + s
        return t, s, tok_off, pos

    def obs_body(k, slot, fire_next, wait_out):
        t, s, tok_off, pos = obs_unit(k)
        wait_gathers(w_obs_hbm, slot)
        if fire_next:
            _, _, nxt_off, _ = obs_unit(k + 1)
            fire(nxt_off, 1024, obs_idx_hbm, w_obs_hbm, 1 - slot)
        if wait_out:
            out_desc(0, slot).wait()
        for c in range(_D // 16):
            bias_v[pl.ds(c * 16, 16)] = (
                pe_obs_v[s, pl.ds(c * 16, 16)]
                + pe_time_v[t, pl.ds(c * 16, 16)]
            )
        transpose_bias_block(slot)
        out_desc(pos, slot).start()

    _, _, off0, _ = obs_unit(0)
    fire(off0, 1024, obs_idx_hbm, w_obs_hbm, 0)

    def obs_pair(kk, carry):
        k0 = 2 * kk

        @pl.when(kk == 0)
        def _():
            obs_body(k0, 0, True, False)
            obs_body(k0 + 1, 1, True, False)

        @pl.when(kk > 0)
        def _():
            obs_body(k0, 0, True, True)
            obs_body(k0 + 1, 1, True, True)

        return carry

    lax.fori_loop(0, (_OBS_PER_W - 1) // 2, obs_pair, 0)
    obs_body(_OBS_PER_W - 1, 0, False, True)
    out_desc(0, 0).wait()
    out_desc(0, 1).wait()

    def act_unit(k):
        i = wid + _NW * k
        t = lax.div(i, _LA)
        a = lax.rem(i, _LA)
        tok_off = t * 1024 + a * 128
        pos = t * _LT + _LO + 1 + a
        return t, a, tok_off, pos

    def act_body(k, slot, fire_next, wait_out):
        t, a, tok_off, pos = act_unit(k)
        wait_gathers(w_act_hbm, slot)
        if fire_next:
            @pl.when(wid + _NW * (k + 1) < _N_ACT)
            def _():
                _, _, nxt_off, _ = act_unit(k + 1)
                fire(nxt_off, 512, act_idx_hbm, w_act_hbm, 1 - slot)
        if wait_out:
            out_desc(0, slot).wait()
        for c in range(_D // 16):
            bias_v[pl.ds(c * 16, 16)] = (
                pe_act_v[a, pl.ds(c * 16, 16)]
                + pe_time_v[t, pl.ds(c * 16, 16)]
            )
        transpose_bias_block(slot)
        out_desc(pos, slot).start()

    _, _, aoff0, _ = act_unit(0)
    fire(aoff0, 512, act_idx_hbm, w_act_hbm, 0)
    act_body(0, 0, True, False)
    act_body(1, 1, True, False)
    act_body(2, 0, True, True)
    act_body(3, 1, True, True)
    act_body(4, 0, True, True)
    act_body(5, 1, True, True)

    @pl.when(wid + _NW * 6 < _N_ACT)
    def _():
        act_body(6, 0, False, True)

    out_desc(0, 0).wait()
    out_desc(0, 1).wait()

    def sp_body(k, slot):
        t = wid + _NW * k
        pos = t * _LT + _LO
        for c in range(_D // 16):
            bias_v[pl.ds(c * 16, 16)] = (
                sp_v[0, pl.ds(c * 16, 16)]
                + pe_time_v[t, pl.ds(c * 16, 16)]
            )
        splat_block(slot)
        out_desc(pos, slot).start()

    sp_body(0, 0)

    @pl.when(wid + _NW < _T)
    def _():
        sp_body(1, 1)

    out_desc(0, 0).wait()

    @pl.when(wid + _NW < _T)
    def _():
        out_desc(0, 1).wait()


def kernel(obs_tokens, act_tokens, W_obs, W_act, W_special,
           PE_obs, PE_act, PE_special, PE_time):
    obs_idx = (obs_tokens.reshape(2, 128, _T, 2, 8)
               .transpose(2, 3, 0, 4, 1).reshape(-1))
    act_idx = (act_tokens.reshape(2, 128, _T, 4)
               .transpose(2, 0, 3, 1).reshape(-1))
    y = _episode_sc(obs_idx, act_idx, W_obs, W_act, W_special,
                    PE_obs, PE_act, PE_special, PE_time)
    out = (y.reshape(_NPOS, 8, 2, 8, 128)
           .transpose(2, 4, 0, 1, 3).reshape(_B, _NPOS, _D))
    return out
